# Initial kernel scaffold; baseline (speedup 1.0000x reference)
import jax
import jax.numpy as jnp
from jax.experimental import pallas as pl


def kernel(x, edge_index, W1, a_src1, a_dst1, b1, W2, a_src2, a_dst2, b2, Wc, bc):
    raise NotImplementedError("write your pallas kernel here")


# speedup vs baseline: 6.7245x; 6.7245x over previous
import functools
import jax
import jax.numpy as jnp
from jax import lax
from jax.experimental import pallas as pl
from jax.experimental.pallas import tpu as pltpu
from jax.experimental.pallas import tpu_sc as plsc

N = 10000
D = 256
HALF = 128
C = 2
E = 160000
NEDGE = E + N
EPAD = 180224
NPAD = 10240
STRIPE = NPAD // 16
A_CHUNKS = 44
B_CHUNKS = 88
B_WIN = 8
CHUNK = 128
ROWBLK = 1000
GRID = N // ROWBLK

_mesh = plsc.VectorSubcoreMesh(core_axis_name="c", subcore_axis_name="s")
_sc_params = pltpu.CompilerParams(needs_layout_passes=False)


def _tc1_body(x_ref, w_ref, asv_ref, adv_ref, hlo_ref, hhi_ref, as_ref, ad_ref,
              ms_ref):
    i = pl.program_id(0)
    h = jnp.dot(x_ref[...], w_ref[...], preferred_element_type=jnp.float32)
    hlo_ref[...] = h[:, :HALF]
    hhi_ref[...] = h[:, HALF:]
    asr = jnp.sum(h * asv_ref[...], axis=1)
    as_ref[...] = asr.reshape(1, 1, ROWBLK)
    ad_ref[...] = jnp.sum(h * adv_ref[...], axis=1).reshape(1, 1, ROWBLK)

    @pl.when(i == 0)
    def _():
        ms_ref[...] = jnp.full((1, 128), -jnp.inf, jnp.float32)
    ms_ref[...] = jnp.maximum(ms_ref[...], jnp.max(asr))


def _tc_mid_body(lo_ref, hi_ref, dn_ref, b_ref, w_ref, asv_ref, adv_ref,
                 hlo_ref, hhi_ref, as_ref, ad_ref, ms_ref):
    inv = 1.0 / jnp.maximum(dn_ref[0, 0, :], 1e-16)
    in_lo = jnp.maximum(lo_ref[0] * inv[:, None] + b_ref[0, :HALF], 0.0)
    in_hi = jnp.maximum(hi_ref[0] * inv[:, None] + b_ref[0, HALF:], 0.0)
    w = w_ref[...]
    h = (jnp.dot(in_lo, w[:HALF, :], preferred_element_type=jnp.float32)
         + jnp.dot(in_hi, w[HALF:, :], preferred_element_type=jnp.float32))
    hlo_ref[...] = h[:, :HALF]
    hhi_ref[...] = h[:, HALF:]
    asr = jnp.sum(h * asv_ref[...], axis=1)
    as_ref[...] = asr.reshape(1, 1, ROWBLK)
    ad_ref[...] = jnp.sum(h * adv_ref[...], axis=1).reshape(1, 1, ROWBLK)

    i = pl.program_id(0)

    @pl.when(i == 0)
    def _():
        ms_ref[...] = jnp.full((1, 128), -jnp.inf, jnp.float32)
    ms_ref[...] = jnp.maximum(ms_ref[...], jnp.max(asr))


def _tc3_body(lo_ref, hi_ref, dn_ref, b_ref, wc_ref, bc_ref, out_ref):
    inv = 1.0 / jnp.maximum(dn_ref[0, 0, :], 1e-16)
    in_lo = jnp.maximum(lo_ref[0] * inv[:, None] + b_ref[0, :HALF], 0.0)
    in_hi = jnp.maximum(hi_ref[0] * inv[:, None] + b_ref[0, HALF:], 0.0)
    wc = wc_ref[...]
    logits = (jnp.dot(in_lo, wc[:HALF, :], preferred_element_type=jnp.float32)
              + jnp.dot(in_hi, wc[HALF:, :], preferred_element_type=jnp.float32)
              + bc_ref[...])
    m = jnp.max(logits, axis=1, keepdims=True)
    lse = m + jnp.log(jnp.sum(jnp.exp(logits - m), axis=1, keepdims=True))
    out_ref[...] = logits - lse


_vec_spec = pl.BlockSpec((1, D), lambda i: (0, 0))
_full_w_spec = pl.BlockSpec((D, D), lambda i: (0, 0))
_half_out_spec = pl.BlockSpec((ROWBLK, HALF), lambda i: (i, 0))
_alpha_out_spec = pl.BlockSpec((1, 1, ROWBLK), lambda i: (i, 0, 0))
_acc_lo_spec = pl.BlockSpec((1, ROWBLK, HALF), lambda i: (0, i, 0))
_acc_hi_spec = pl.BlockSpec((1, ROWBLK, HALF), lambda i: (1, i, 0))
_dn_spec = pl.BlockSpec((1, 1, ROWBLK), lambda i: (i, 0, 0))

_tc1 = pl.pallas_call(
    _tc1_body,
    grid=(GRID,),
    in_specs=[pl.BlockSpec((ROWBLK, D), lambda i: (i, 0)),
              _full_w_spec, _vec_spec, _vec_spec],
    out_specs=[_half_out_spec, _half_out_spec, _alpha_out_spec, _alpha_out_spec,
               pl.BlockSpec((1, 128), lambda i: (0, 0))],
    out_shape=[jax.ShapeDtypeStruct((N, HALF), jnp.float32),
               jax.ShapeDtypeStruct((N, HALF), jnp.float32),
               jax.ShapeDtypeStruct((GRID, 1, ROWBLK), jnp.float32),
               jax.ShapeDtypeStruct((GRID, 1, ROWBLK), jnp.float32),
               jax.ShapeDtypeStruct((1, 128), jnp.float32)],
)

_tc_mid = pl.pallas_call(
    _tc_mid_body,
    grid=(GRID,),
    in_specs=[_acc_lo_spec, _acc_hi_spec, _dn_spec, _vec_spec,
              _full_w_spec, _vec_spec, _vec_spec],
    out_specs=[_half_out_spec, _half_out_spec, _alpha_out_spec, _alpha_out_spec,
               pl.BlockSpec((1, 128), lambda i: (0, 0))],
    out_shape=[jax.ShapeDtypeStruct((N, HALF), jnp.float32),
               jax.ShapeDtypeStruct((N, HALF), jnp.float32),
               jax.ShapeDtypeStruct((GRID, 1, ROWBLK), jnp.float32),
               jax.ShapeDtypeStruct((GRID, 1, ROWBLK), jnp.float32),
               jax.ShapeDtypeStruct((1, 128), jnp.float32)],
)

_tc3 = pl.pallas_call(
    _tc3_body,
    grid=(GRID,),
    in_specs=[_acc_lo_spec, _acc_hi_spec, _dn_spec, _vec_spec,
              pl.BlockSpec((D, C), lambda i: (0, 0)),
              pl.BlockSpec((1, C), lambda i: (0, 0))],
    out_specs=pl.BlockSpec((ROWBLK, C), lambda i: (i, 0)),
    out_shape=jax.ShapeDtypeStruct((N, C), jnp.float32),
)


def _edge_a_body(asrc_hbm, adst_hbm, ms_hbm, src_hbm, dst_hbm, w_hbm, dpart_hbm,
                 asrc_v, adst_v, msb, srcb, dstb, wb, zb, denom_sh):
    c = lax.axis_index("c")
    s = lax.axis_index("s")
    wid = s * 2 + c

    pltpu.sync_copy(asrc_hbm, asrc_v)
    pltpu.sync_copy(adst_hbm, adst_v)
    pltpu.sync_copy(ms_hbm, msb)
    pltpu.sync_copy(src_hbm.at[wid], srcb)
    pltpu.sync_copy(dst_hbm.at[wid], dstb)

    zeros16 = jnp.zeros((16,), jnp.float32)

    def _zb(i, _):
        zb[pl.ds(i * 16, 16)] = zeros16
        return 0
    lax.fori_loop(0, STRIPE // 16, _zb, 0)
    pltpu.sync_copy(zb, denom_sh.at[pl.ds(s * STRIPE, STRIPE)])
    plsc.subcore_barrier()

    ms = msb[pl.ds(0, 16)]

    def _chunk(j, _):
        for k in range(CHUNK // 16):
            sl = pl.ds(k * 16, 16)
            sv = srcb[j, sl]
            dv = dstb[j, sl]
            a_s = plsc.load_gather(asrc_v, [sv])
            a_d = plsc.load_gather(adst_v, [dv])
            x = a_s + a_d
            e = jnp.where(x >= 0, x, 0.2 * x)
            y = ms + a_d
            kd = jnp.where(y >= 0, y, 0.2 * y)
            w = jnp.exp(e - kd)
            gid = wid * (A_CHUNKS * CHUNK) + j * CHUNK + k * 16 + lax.iota(jnp.int32, 16)
            wb[j, sl] = jnp.where(gid < NEDGE, w, 0.0)
        pltpu.sync_copy(wb.at[j], denom_sh.at[dstb.at[j]], add=True)
        return 0

    lax.fori_loop(0, A_CHUNKS, _chunk, 0)
    pltpu.sync_copy(wb, w_hbm.at[wid])
    plsc.subcore_barrier()
    pltpu.sync_copy(denom_sh.at[pl.ds(s * STRIPE, STRIPE)],
                    dpart_hbm.at[c, pl.ds(s * STRIPE, STRIPE)])


_edge_a = pl.kernel(
    _edge_a_body,
    out_type=[jax.ShapeDtypeStruct((32, A_CHUNKS, CHUNK), jnp.float32),
              jax.ShapeDtypeStruct((2, NPAD), jnp.float32)],
    mesh=_mesh,
    compiler_params=_sc_params,
    scratch_types=[
        pltpu.VMEM((N,), jnp.float32),
        pltpu.VMEM((N,), jnp.float32),
        pltpu.VMEM((16,), jnp.float32),
        pltpu.VMEM((A_CHUNKS, CHUNK), jnp.int32),
        pltpu.VMEM((A_CHUNKS, CHUNK), jnp.int32),
        pltpu.VMEM((A_CHUNKS, CHUNK), jnp.float32),
        pltpu.VMEM((STRIPE,), jnp.float32),
        pltpu.VMEM_SHARED((NPAD,), jnp.float32),
    ],
)


def _edge_b_body(hlo_hbm, hhi_hbm, w_hbm, src_hbm, dst_hbm, out_hbm,
                 srcb, dstb, wb, buf, acc_sh, sem):
    c = lax.axis_index("c")
    s = lax.axis_index("s")

    zeros16 = jnp.zeros((16,), jnp.float32)

    def _zrow(r, _):
        for k in range(HALF // 16):
            buf[r, pl.ds(k * 16, 16)] = zeros16
        return 0
    lax.fori_loop(0, CHUNK, _zrow, 0)

    def _zcp(p, _):
        pltpu.sync_copy(buf, acc_sh.at[pl.ds(s * STRIPE + p * CHUNK, CHUNK)])
        return 0
    lax.fori_loop(0, STRIPE // CHUNK, _zcp, 0)
    plsc.subcore_barrier()

    def _run(h_hbm):
        def _window(win, _):
            base = win * B_WIN
            pltpu.sync_copy(src_hbm.at[s, pl.ds(base, B_WIN)], srcb)
            pltpu.sync_copy(dst_hbm.at[s, pl.ds(base, B_WIN)], dstb)
            pltpu.sync_copy(w_hbm.at[s, pl.ds(base, B_WIN)], wb)

            def _chunk(j, _):
                pltpu.async_copy(h_hbm.at[srcb.at[j]], buf, sem).wait()

                def _grp(g, _):
                    wv = wb[j, pl.ds(g * 16, 16)]
                    for r16 in range(16):
                        row = g * 16 + r16
                        wsc = wv[r16]
                        for k in range(HALF // 16):
                            sl = pl.ds(k * 16, 16)
                            buf[row, sl] = buf[row, sl] * wsc
                    return 0
                lax.fori_loop(0, CHUNK // 16, _grp, 0)
                pltpu.sync_copy(buf, acc_sh.at[dstb.at[j]], add=True)
                return 0
            lax.fori_loop(0, B_WIN, _chunk, 0)
            return 0
        lax.fori_loop(0, B_CHUNKS // B_WIN, _window, 0)

    @pl.when(c == 0)
    def _():
        _run(hlo_hbm)

    @pl.when(c == 1)
    def _():
        _run(hhi_hbm)

    plsc.subcore_barrier()
    pltpu.sync_copy(acc_sh.at[pl.ds(s * STRIPE, STRIPE)],
                    out_hbm.at[c, pl.ds(s * STRIPE, STRIPE)])


_edge_b = pl.kernel(
    _edge_b_body,
    out_type=jax.ShapeDtypeStruct((2, NPAD, HALF), jnp.float32),
    mesh=_mesh,
    compiler_params=_sc_params,
    scratch_types=[
        pltpu.VMEM((B_WIN, CHUNK), jnp.int32),
        pltpu.VMEM((B_WIN, CHUNK), jnp.int32),
        pltpu.VMEM((B_WIN, CHUNK), jnp.float32),
        pltpu.VMEM((CHUNK, HALF), jnp.float32),
        pltpu.VMEM_SHARED((NPAD, HALF), jnp.float32),
        pltpu.SemaphoreType.DMA,
    ],
)


def _gat_layer(hlo, hhi, asrc, adst, msv, src_a, dst_a, src_b, dst_b):
    w_pad, dparts = _edge_a(asrc, adst, msv, src_a, dst_a)
    w_b = w_pad.reshape(16, B_CHUNKS, CHUNK)
    acc = _edge_b(hlo, hhi, w_b, src_b, dst_b)
    dn = (dparts[0, :N] + dparts[1, :N]).reshape(GRID, 1, ROWBLK)
    return acc, dn


@jax.jit
def kernel(x, edge_index, W1, a_src1, a_dst1, b1, W2, a_src2, a_dst2, b2, Wc, bc):
    loop = jnp.arange(N, dtype=edge_index.dtype)
    pad = jnp.zeros((EPAD - NEDGE,), edge_index.dtype)
    src = jnp.concatenate([edge_index[0], loop, pad])
    dst = jnp.concatenate([edge_index[1], loop, pad])
    src_a = src.reshape(32, A_CHUNKS, CHUNK)
    dst_a = dst.reshape(32, A_CHUNKS, CHUNK)
    src_b = src.reshape(16, B_CHUNKS, CHUNK)
    dst_b = dst.reshape(16, B_CHUNKS, CHUNK)

    h1lo, h1hi, as1, ad1, ms1 = _tc1(x, W1, a_src1.reshape(1, D),
                                     a_dst1.reshape(1, D))
    acc1, dn1 = _gat_layer(h1lo, h1hi, as1.reshape(N), ad1.reshape(N),
                           ms1[0, :16], src_a, dst_a, src_b, dst_b)
    h2lo, h2hi, as2, ad2, ms2 = _tc_mid(acc1, acc1, dn1, b1.reshape(1, D), W2,
                                        a_src2.reshape(1, D),
                                        a_dst2.reshape(1, D))
    acc2, dn2 = _gat_layer(h2lo, h2hi, as2.reshape(N), ad2.reshape(N),
                           ms2[0, :16], src_a, dst_a, src_b, dst_b)
    return _tc3(acc2, acc2, dn2, b2.reshape(1, D), Wc, bc.reshape(1, C))
